# baseline (speedup 1.0000x reference)
import functools

import jax
import jax.numpy as jnp
from jax import lax
from jax.experimental import pallas as pl
from jax.experimental.pallas import tpu as pltpu
from jax.experimental.pallas import tpu_sc as plsc

S = 8192
D = 1024
NC = 2
NS = 8
NW = NC * NS
ROWS = S // NW
SIZES = (32,) * 16
OFFS = tuple(sum(SIZES[:i]) for i in range(len(SIZES)))
NB = 3
BUF = max(SIZES)
AHEAD = 3
NCHUNK = len(SIZES)
assert sum(SIZES) == ROWS

_mesh = plsc.VectorSubcoreMesh(core_axis_name="c", subcore_axis_name="s", num_subcores=NS)


@functools.partial(
    pl.kernel,
    mesh=_mesh,
    out_type=jax.ShapeDtypeStruct((S, D), jnp.float32),
    scratch_types=(
        [pltpu.VMEM((BUF, D), jnp.float32) for _ in range(NB)]
        + [pltpu.SemaphoreType.DMA for _ in range(2 * NB)]
    ),
)
def _copy_table(table_hbm, out_hbm, *scratch):
    bufs = scratch[:NB]
    sin = scratch[NB:2 * NB]
    sout = scratch[2 * NB:]
    wid = lax.axis_index("s") * NC + lax.axis_index("c")
    base = wid * ROWS

    def start_in(g):
        return pltpu.async_copy(
            table_hbm.at[pl.ds(base + OFFS[g], SIZES[g])],
            bufs[g % NB].at[pl.ds(0, SIZES[g])],
            sin[g % NB],
        )

    def start_out(g):
        return pltpu.async_copy(
            bufs[g % NB].at[pl.ds(0, SIZES[g])],
            out_hbm.at[pl.ds(base + OFFS[g], SIZES[g])],
            sout[g % NB],
        )

    cin = [None] * NCHUNK
    cout = [None] * NCHUNK
    for g in range(min(AHEAD, NCHUNK)):
        cin[g] = start_in(g)
    waited = set()
    for g in range(NCHUNK):
        cin[g].wait()
        cout[g] = start_out(g)
        n = g + AHEAD
        if n < NCHUNK:
            if n - NB >= 0:
                cout[n - NB].wait()
                waited.add(n - NB)
            cin[n] = start_in(n)
    for g in range(NCHUNK):
        if g not in waited:
            cout[g].wait()


def kernel(inputs, table):
    del inputs
    return _copy_table(table)[None]


# speedup vs baseline: 1.0879x; 1.0879x over previous
import functools

import jax
import jax.numpy as jnp
from jax import lax
from jax.experimental import pallas as pl
from jax.experimental.pallas import tpu as pltpu
from jax.experimental.pallas import tpu_sc as plsc

S = 8192
D = 1024
R_TC = 2048
NC = 2
NS = 16
NW = NC * NS
ROWS = (S - R_TC) // NW
SIZES = (32,) * (ROWS // 32)
OFFS = tuple(sum(SIZES[:i]) for i in range(len(SIZES)))
NB = 3
BUF = max(SIZES)
AHEAD = 3
NCHUNK = len(SIZES)
assert sum(SIZES) == ROWS

BLK = 512

_mesh = plsc.VectorSubcoreMesh(core_axis_name="c", subcore_axis_name="s")


@functools.partial(
    pl.kernel,
    mesh=_mesh,
    out_type=jax.ShapeDtypeStruct((S, D), jnp.float32),
    scratch_types=(
        [pltpu.VMEM((BUF, D), jnp.float32) for _ in range(NB)]
        + [pltpu.SemaphoreType.DMA for _ in range(2 * NB)]
    ),
)
def _copy_tail_sc(table_hbm, out_hbm, *scratch):
    bufs = scratch[:NB]
    sin = scratch[NB:2 * NB]
    sout = scratch[2 * NB:]
    wid = lax.axis_index("s") * NC + lax.axis_index("c")
    base = R_TC + wid * ROWS

    def start_in(g):
        return pltpu.async_copy(
            table_hbm.at[pl.ds(base + OFFS[g], SIZES[g])],
            bufs[g % NB].at[pl.ds(0, SIZES[g])],
            sin[g % NB],
        )

    def start_out(g):
        return pltpu.async_copy(
            bufs[g % NB].at[pl.ds(0, SIZES[g])],
            out_hbm.at[pl.ds(base + OFFS[g], SIZES[g])],
            sout[g % NB],
        )

    cin = [None] * NCHUNK
    cout = [None] * NCHUNK
    for g in range(min(AHEAD, NCHUNK)):
        cin[g] = start_in(g)
    waited = set()
    for g in range(NCHUNK):
        cin[g].wait()
        cout[g] = start_out(g)
        n = g + AHEAD
        if n < NCHUNK:
            if n - NB >= 0:
                cout[n - NB].wait()
                waited.add(n - NB)
            cin[n] = start_in(n)
    for g in range(NCHUNK):
        if g not in waited:
            cout[g].wait()


def _tc_body(table_ref, part_ref, out_ref):
    del part_ref
    out_ref[...] = table_ref[...]


def kernel(inputs, table):
    del inputs
    part = _copy_tail_sc(table)
    out = pl.pallas_call(
        _tc_body,
        grid=(R_TC // BLK,),
        in_specs=[
            pl.BlockSpec((BLK, D), lambda i: (i, 0)),
            pl.BlockSpec(memory_space=pltpu.MemorySpace.HBM),
        ],
        out_specs=pl.BlockSpec((BLK, D), lambda i: (i, 0)),
        out_shape=jax.ShapeDtypeStruct((S, D), jnp.float32),
        input_output_aliases={1: 0},
    )(table, part)
    return out[None]


# speedup vs baseline: 1.1418x; 1.0496x over previous
import functools

import jax
import jax.numpy as jnp
from jax import lax
from jax.experimental import pallas as pl
from jax.experimental.pallas import tpu as pltpu
from jax.experimental.pallas import tpu_sc as plsc

S = 8192
D = 1024
NC = 2
NS = 16
NW = NC * NS
ROWS = S // NW
CH = 32
NB = 3
AHEAD = 3
NCHUNK = ROWS // CH

_mesh = plsc.VectorSubcoreMesh(core_axis_name="c", subcore_axis_name="s")


@functools.partial(
    pl.kernel,
    mesh=_mesh,
    out_type=jax.ShapeDtypeStruct((S, D), jnp.float32),
    scratch_types=(
        [pltpu.VMEM((CH, D), jnp.float32) for _ in range(NB)]
        + [pltpu.SemaphoreType.DMA for _ in range(2 * NB)]
    ),
)
def _copy_table(table_hbm, out_hbm, *scratch):
    bufs = scratch[:NB]
    sin = scratch[NB:2 * NB]
    sout = scratch[2 * NB:]
    wid = lax.axis_index("s") * NC + lax.axis_index("c")
    base = wid * ROWS

    def start_in(g):
        return pltpu.async_copy(
            table_hbm.at[pl.ds(base + g * CH, CH)], bufs[g % NB], sin[g % NB]
        )

    def start_out(g):
        return pltpu.async_copy(
            bufs[g % NB], out_hbm.at[pl.ds(base + g * CH, CH)], sout[g % NB]
        )

    cin = [None] * NCHUNK
    cout = [None] * NCHUNK
    for g in range(min(AHEAD, NCHUNK)):
        cin[g] = start_in(g)
    waited = set()
    for g in range(NCHUNK):
        cin[g].wait()
        cout[g] = start_out(g)
        n = g + AHEAD
        if n < NCHUNK:
            if n - NB >= 0:
                cout[n - NB].wait()
                waited.add(n - NB)
            cin[n] = start_in(n)
    for g in range(NCHUNK):
        if g not in waited:
            cout[g].wait()


def kernel(inputs, table):
    del inputs
    return _copy_table(table)[None]
